# Initial kernel scaffold; baseline (speedup 1.0000x reference)
import jax
import jax.numpy as jnp
from jax.experimental import pallas as pl


def kernel(x, edge_index, desc, params):
    raise NotImplementedError("write your pallas kernel here")


# speedup vs baseline: 4.4654x; 4.4654x over previous
import functools

import jax
import jax.numpy as jnp
from jax import lax
from jax.experimental import pallas as pl
from jax.experimental.pallas import tpu as pltpu
from jax.experimental.pallas import tpu_sc as plsc

_N, _E, _D, _H, _EXTRA, _NCLS = 10000, 320000, 128, 128, 16, 10
_LAYERS = 3

_SC_CORES = 2
_SC_SUBCORES = 16
_NW = _SC_CORES * _SC_SUBCORES
_EPW = _E // _NW
_EB = 80
_NB = _EPW // _EB
_ZB = 400
_NZB = _N // _ZB


def _agg_body(h_hbm, src_hbm, dst_hbm, zero_hbm, out_hbm,
              src_v, dst_v, rows_v, acc_sh, sem):
    cid = lax.axis_index("c")
    sid = lax.axis_index("s")
    wid = sid * _SC_CORES + cid

    def zstep(k, carry):
        blk = sid + _SC_SUBCORES * k

        @pl.when(blk < _NZB)
        def _():
            off = pl.multiple_of(blk * _ZB, 8)
            pltpu.sync_copy(zero_hbm.at[pl.ds(off, _ZB)],
                            acc_sh.at[pl.ds(off, _ZB)])
        return carry

    lax.fori_loop(0, (_NZB + _SC_SUBCORES - 1) // _SC_SUBCORES, zstep, 0)
    plsc.subcore_barrier()
    base = wid * _EPW

    def step(i, carry):
        off = pl.multiple_of(base + i * _EB, 8)
        pltpu.sync_copy(src_hbm.at[pl.ds(off, _EB)], src_v)
        pltpu.sync_copy(dst_hbm.at[pl.ds(off, _EB)], dst_v)
        pltpu.async_copy(h_hbm.at[src_v], rows_v, sem).wait()
        pltpu.sync_copy(rows_v, acc_sh.at[dst_v], add=True)
        return carry

    lax.fori_loop(0, _NB, step, 0)
    plsc.subcore_barrier()

    def ostep(k, carry):
        blk = sid + _SC_SUBCORES * k

        @pl.when(blk < _NZB)
        def _():
            off = pl.multiple_of(blk * _ZB, 8)
            pltpu.sync_copy(acc_sh.at[pl.ds(off, _ZB)],
                            out_hbm.at[cid, pl.ds(off, _ZB)])
        return carry

    lax.fori_loop(0, (_NZB + _SC_SUBCORES - 1) // _SC_SUBCORES, ostep, 0)


_agg = pl.kernel(
    _agg_body,
    out_type=jax.ShapeDtypeStruct((_SC_CORES, _N, _D), jnp.float32),
    mesh=plsc.VectorSubcoreMesh(core_axis_name="c", subcore_axis_name="s"),
    scratch_types=[
        pltpu.VMEM((_EB,), jnp.int32),
        pltpu.VMEM((_EB,), jnp.int32),
        pltpu.VMEM((_EB, _D), jnp.float32),
        pltpu.VMEM_SHARED((_N, _D), jnp.float32),
        pltpu.SemaphoreType.DMA,
    ],
)


def _ln(z, g, b):
    m = jnp.mean(z, axis=-1, keepdims=True)
    v = jnp.mean((z - m) * (z - m), axis=-1, keepdims=True)
    return (z - m) * lax.rsqrt(v + 1e-5) * g + b


_BR = 1000


def _mlp_body(h_ref, a0_ref, a1_ref, eps_ref, w1_ref, b1_ref, g1_ref,
              be1_ref, w2_ref, b2_ref, gn_ref, bn_ref, out_ref):
    rst = eps_ref[0, 0] * h_ref[...] + a0_ref[...] + a1_ref[...]
    z = jnp.dot(rst, w1_ref[...], preferred_element_type=jnp.float32) + b1_ref[...]
    z = jnp.maximum(_ln(z, g1_ref[...], be1_ref[...]), 0.0)
    z = jnp.dot(z, w2_ref[...], preferred_element_type=jnp.float32) + b2_ref[...]
    out_ref[...] = jnp.maximum(_ln(z, gn_ref[...], bn_ref[...]), 0.0)


def _bcast(shape):
    return pl.BlockSpec(shape, lambda i: (0, 0))


_mlp = pl.pallas_call(
    _mlp_body,
    grid=(_N // _BR,),
    in_specs=[
        pl.BlockSpec((_BR, _D), lambda i: (i, 0)),
        pl.BlockSpec((_BR, _D), lambda i: (i, 0)),
        pl.BlockSpec((_BR, _D), lambda i: (i, 0)),
        _bcast((1, 1)),
        _bcast((_D, _H)),
        _bcast((1, _H)),
        _bcast((1, _H)),
        _bcast((1, _H)),
        _bcast((_H, _H)),
        _bcast((1, _H)),
        _bcast((1, _H)),
        _bcast((1, _H)),
    ],
    out_specs=pl.BlockSpec((_BR, _H), lambda i: (i, 0)),
    out_shape=jax.ShapeDtypeStruct((_N, _H), jnp.float32),
)


def _head_body(h_ref, desc_ref, wh_ref, wd_ref, b1_ref, g_ref, be_ref,
               w2_ref, b2_ref, out_ref):
    hg = jnp.sum(h_ref[...], axis=0, keepdims=True)
    y = (jnp.dot(hg, wh_ref[...], preferred_element_type=jnp.float32)
         + jnp.dot(desc_ref[...], wd_ref[...], preferred_element_type=jnp.float32)
         + b1_ref[...])
    y = jnp.maximum(_ln(y, g_ref[...], be_ref[...]), 0.0)
    out_ref[...] = jnp.dot(y, w2_ref[...], preferred_element_type=jnp.float32) + b2_ref[...]


_head = pl.pallas_call(
    _head_body,
    out_shape=jax.ShapeDtypeStruct((1, _NCLS), jnp.float32),
)


def kernel(x, edge_index, desc, params):
    src = edge_index[0]
    dst = edge_index[1]
    zeros_nd = jnp.zeros((_N, _D), jnp.float32)
    h = x
    for l in range(_LAYERS):
        agg = _agg(h, src, dst, zeros_nd)
        eps1 = (1.0 + params['eps_%d' % l]).reshape(1, 1)
        h = _mlp(h, agg[0], agg[1], eps1,
                 params['W1_%d' % l], params['b1_%d' % l].reshape(1, _H),
                 params['g1_%d' % l].reshape(1, _H),
                 params['be1_%d' % l].reshape(1, _H),
                 params['W2_%d' % l], params['b2_%d' % l].reshape(1, _H),
                 params['gn_%d' % l].reshape(1, _H),
                 params['bn_%d' % l].reshape(1, _H))
    out = _head(h, desc,
                params['fc1_W'][:_H], params['fc1_W'][_H:],
                params['fc1_b'].reshape(1, _H),
                params['n1_g'].reshape(1, _H), params['n1_b'].reshape(1, _H),
                params['fc2_W'], params['fc2_b'].reshape(1, _NCLS))
    return out
